# baseline (speedup 1.0000x reference)
import functools

import jax
import jax.numpy as jnp
from jax import lax
from jax.experimental import pallas as pl
from jax.experimental.pallas import tpu as pltpu
from jax.experimental.pallas import tpu_sc as plsc

N = 10000
E = 320000
F = 128
H = F // 2
NC = 2
NS = 16
L = 16
EPT = E // NS
B = 80
NBLK = EPT // B
ROWS = N // NS

_mesh = plsc.VectorSubcoreMesh(core_axis_name="c", subcore_axis_name="s")

_SPLAT_DNUMS = lax.GatherDimensionNumbers(
    offset_dims=(), collapsed_slice_dims=(0,), start_index_map=(0,))


def _splat(vec, j):
    idx = jnp.full((L, 1), j, jnp.int32)
    return lax.gather(vec, idx, _SPLAT_DNUMS, (1,),
                      mode=lax.GatherScatterMode.PROMISE_IN_BOUNDS)


W = H // 2


def _mlp_body(q_ref, mu_ref, w1_ref, b1_ref, w2_ref, b2_ref,
              xa_ref, xb0_ref, xb1_ref, qh_ref, mq0_ref, mq1_ref):
    h = jnp.dot(q_ref[...], w1_ref[...], preferred_element_type=jnp.float32)
    h = h + b1_ref[...]
    h = h * lax.logistic(h)
    x = jnp.dot(h, w2_ref[...], preferred_element_type=jnp.float32)
    x = x + b2_ref[...]
    for c in range(NC):
        xa_ref[c] = x[:, c * H:(c + 1) * H]
        qh_ref[c] = q_ref[:, c * H:(c + 1) * H]
        for p, (xb_ref, mq_ref) in enumerate(((xb0_ref, mq0_ref),
                                              (xb1_ref, mq1_ref))):
            c0 = c * H + p * W
            xb_ref[c] = jnp.concatenate(
                [x[:, F + c0:F + c0 + W], x[:, 2 * F + c0:2 * F + c0 + W]],
                axis=1)
            mq_ref[c] = mu_ref[:, :, c0:c0 + W]


def _mlp(q2, mu, W1, b1, W2, b2):
    blk = 1000
    grid = N // blk
    return pl.pallas_call(
        _mlp_body,
        grid=(grid,),
        in_specs=[
            pl.BlockSpec((blk, F), lambda i: (i, 0)),
            pl.BlockSpec((blk, 3, F), lambda i: (i, 0, 0)),
            pl.BlockSpec((F, F), lambda i: (0, 0)),
            pl.BlockSpec((1, F), lambda i: (0, 0)),
            pl.BlockSpec((F, 3 * F), lambda i: (0, 0)),
            pl.BlockSpec((1, 3 * F), lambda i: (0, 0)),
        ],
        out_specs=[
            pl.BlockSpec((NC, blk, H), lambda i: (0, i, 0)),
            pl.BlockSpec((NC, blk, 2 * W), lambda i: (0, i, 0)),
            pl.BlockSpec((NC, blk, 2 * W), lambda i: (0, i, 0)),
            pl.BlockSpec((NC, blk, H), lambda i: (0, i, 0)),
            pl.BlockSpec((NC, blk, 3, W), lambda i: (0, i, 0, 0)),
            pl.BlockSpec((NC, blk, 3, W), lambda i: (0, i, 0, 0)),
        ],
        out_shape=[
            jax.ShapeDtypeStruct((NC, N, H), jnp.float32),
            jax.ShapeDtypeStruct((NC, N, 2 * W), jnp.float32),
            jax.ShapeDtypeStruct((NC, N, 2 * W), jnp.float32),
            jax.ShapeDtypeStruct((NC, N, H), jnp.float32),
            jax.ShapeDtypeStruct((NC, N, 3, W), jnp.float32),
            jax.ShapeDtypeStruct((NC, N, 3, W), jnp.float32),
        ],
    )(q2, mu, W1, b1, W2, b2)


NBUF = 3


@functools.partial(
    pl.kernel,
    out_type=jax.ShapeDtypeStruct((N, F), jnp.float32),
    mesh=_mesh,
    compiler_params=pltpu.CompilerParams(use_tc_tiling_on_sc=False),
    scratch_types=[
        pltpu.VMEM_SHARED((N, H), jnp.float32),
        pltpu.VMEM((4, B), jnp.int32),
        pltpu.VMEM((NBUF, B), jnp.int32),
        pltpu.VMEM((NBUF, B, H), jnp.float32),
        pltpu.VMEM((2, B, H), jnp.float32),
        pltpu.VMEM((NBUF, B, H), jnp.float32),
        pltpu.SemaphoreType.DMA((NBUF,)),
        pltpu.SemaphoreType.DMA((2,)),
        pltpu.SemaphoreType.DMA((4,)),
    ],
)
def _dq_phase(wij_hbm, xa_hbm, qh_hbm, idxi_hbm, idxj_hbm, qout_hbm,
              acc, idxi_v, idxj_v, w_v, xg_v, msg_v, lsem, gsem, ssem):
    c = lax.axis_index("c")
    s = lax.axis_index("s")
    r0 = s * ROWS
    pltpu.sync_copy(qh_hbm.at[c].at[pl.ds(r0, ROWS)], acc.at[pl.ds(r0, ROWS)])
    plsc.subcore_barrier()

    e_base = s * EPT
    c0 = c * H

    def lin_copies(b):
        e0 = e_base + b * B
        t = lax.rem(b, NBUF)
        return t, [
            (idxi_hbm.at[pl.ds(e0, B)], idxi_v.at[lax.rem(b, 4)]),
            (idxj_hbm.at[pl.ds(e0, B)], idxj_v.at[t]),
            (wij_hbm.at[pl.ds(e0, B), pl.ds(c0, H)], w_v.at[t]),
        ]

    def start_linear(b):
        t, copies = lin_copies(b)
        for src, dst in copies:
            pltpu.async_copy(src, dst, lsem.at[t])

    def wait_linear(b):
        t, copies = lin_copies(b)
        for src, dst in copies:
            pltpu.make_async_copy(src, dst, lsem.at[t]).wait()

    def start_gather(b):
        pltpu.async_copy(xa_hbm.at[c].at[idxj_v.at[lax.rem(b, NBUF)]],
                         xg_v.at[lax.rem(b, 2)], gsem.at[lax.rem(b, 2)])

    def wait_gather(b):
        pltpu.make_async_copy(xa_hbm.at[c].at[idxj_v.at[lax.rem(b, NBUF)]],
                              xg_v.at[lax.rem(b, 2)],
                              gsem.at[lax.rem(b, 2)]).wait()

    def start_scatter(b):
        pltpu.async_copy(msg_v.at[lax.rem(b, NBUF)],
                         acc.at[idxi_v.at[lax.rem(b, 4)]],
                         ssem.at[lax.rem(b, 4)], add=True)

    def wait_scatter(b):
        pltpu.make_async_copy(msg_v.at[lax.rem(b, NBUF)],
                              acc.at[idxi_v.at[lax.rem(b, 4)]],
                              ssem.at[lax.rem(b, 4)]).wait()

    start_linear(0)
    start_linear(1)
    wait_linear(0)
    start_gather(0)

    @pl.loop(0, NBLK)
    def _blk(b):
        t = lax.rem(b, NBUF)
        tg = lax.rem(b, 2)

        @pl.when(b >= 2)
        def _():
            wait_scatter(b - 2)

        @pl.when(b + 2 < NBLK)
        def _():
            start_linear(b + 2)

        @pl.when(b + 1 < NBLK)
        def _():
            wait_linear(b + 1)
            start_gather(b + 1)

        wait_gather(b)

        @plsc.parallel_loop(0, B)
        def _edge(e):
            for v in range(H // L):
                sl = pl.ds(v * L, L)
                msg_v[t, e, sl] = w_v[t, e, sl] * xg_v[tg, e, sl]

        start_scatter(b)

    wait_scatter(NBLK - 2)
    wait_scatter(NBLK - 1)
    plsc.subcore_barrier()
    pltpu.sync_copy(acc.at[pl.ds(r0, ROWS)],
                    qout_hbm.at[pl.ds(r0, ROWS), pl.ds(c0, H)])


@functools.partial(
    pl.kernel,
    out_type=jax.ShapeDtypeStruct((N, 3, NC, 2, W), jnp.float32),
    mesh=_mesh,
    compiler_params=pltpu.CompilerParams(use_tc_tiling_on_sc=False),
    scratch_types=[
        pltpu.VMEM_SHARED((N, 3, W), jnp.float32),
        pltpu.VMEM((4, B), jnp.int32),
        pltpu.VMEM((NBUF, B), jnp.int32),
        pltpu.VMEM((NBUF, B, W), jnp.float32),
        pltpu.VMEM((NBUF, B, W), jnp.float32),
        pltpu.VMEM((2, B, 2 * W), jnp.float32),
        pltpu.VMEM((2, B, 3, W), jnp.float32),
        pltpu.VMEM((NBUF, B, 3, W), jnp.float32),
        pltpu.VMEM((NBUF, 3, B), jnp.float32),
        pltpu.SemaphoreType.DMA((NBUF,)),
        pltpu.SemaphoreType.DMA((2,)),
        pltpu.SemaphoreType.DMA((4,)),
    ],
)
def _dmu_kernel(wij_hbm, xb0_hbm, mq0_hbm, xb1_hbm, mq1_hbm, dir_hbm,
                idxi_hbm, idxj_hbm, muout_hbm, acc, idxi_v, idxj_v, wm_v,
                wh_v, xg_v, mu_v, msg_v, dir_v, lsem, gsem, ssem):
    c = lax.axis_index("c")
    s = lax.axis_index("s")
    r0 = s * ROWS
    e_base = s * EPT

    for p, (xbp_hbm, muq_hbm) in enumerate(((xb0_hbm, mq0_hbm),
                                            (xb1_hbm, mq1_hbm))):
        c0 = c * H + p * W
        pltpu.sync_copy(muq_hbm.at[c].at[pl.ds(r0, ROWS)],
                        acc.at[pl.ds(r0, ROWS)])
        plsc.subcore_barrier()

        def lin_copies(b):
            e0 = e_base + b * B
            t = lax.rem(b, NBUF)
            ti = lax.rem(b, 4)
            return t, [
                (idxi_hbm.at[pl.ds(e0, B)], idxi_v.at[ti]),
                (idxj_hbm.at[pl.ds(e0, B)], idxj_v.at[t]),
                (wij_hbm.at[pl.ds(e0, B), pl.ds(F + c0, W)], wm_v.at[t]),
                (wij_hbm.at[pl.ds(e0, B), pl.ds(2 * F + c0, W)], wh_v.at[t]),
                (dir_hbm.at[:, pl.ds(e0, B)], dir_v.at[t]),
            ]

        def start_linear(b):
            t, copies = lin_copies(b)
            for src, dst in copies:
                pltpu.async_copy(src, dst, lsem.at[t])

        def wait_linear(b):
            t, copies = lin_copies(b)
            for src, dst in copies:
                pltpu.make_async_copy(src, dst, lsem.at[t]).wait()

        def gat_copies(b):
            t = lax.rem(b, NBUF)
            tg = lax.rem(b, 2)
            return tg, [
                (xbp_hbm.at[c].at[idxj_v.at[t]], xg_v.at[tg]),
                (muq_hbm.at[c].at[idxj_v.at[t]], mu_v.at[tg]),
            ]

        def start_gather(b):
            tg, copies = gat_copies(b)
            for src, dst in copies:
                pltpu.async_copy(src, dst, gsem.at[tg])

        def wait_gather(b):
            tg, copies = gat_copies(b)
            for src, dst in copies:
                pltpu.make_async_copy(src, dst, gsem.at[tg]).wait()

        def start_scatter(b):
            pltpu.async_copy(msg_v.at[lax.rem(b, NBUF)],
                             acc.at[idxi_v.at[lax.rem(b, 4)]],
                             ssem.at[lax.rem(b, 4)], add=True)

        def wait_scatter(b):
            pltpu.make_async_copy(msg_v.at[lax.rem(b, NBUF)],
                                  acc.at[idxi_v.at[lax.rem(b, 4)]],
                                  ssem.at[lax.rem(b, 4)]).wait()

        start_linear(0)
        start_linear(1)
        wait_linear(0)
        start_gather(0)

        @pl.loop(0, NBLK)
        def _blk(b):
            t = lax.rem(b, NBUF)
            tg = lax.rem(b, 2)
            tm = t

            @pl.when(b >= 2)
            def _():
                wait_scatter(b - 2)

            @pl.when(b + 2 < NBLK)
            def _():
                start_linear(b + 2)

            @pl.when(b + 1 < NBLK)
            def _():
                wait_linear(b + 1)
                start_gather(b + 1)

            wait_gather(b)

            @plsc.parallel_loop(0, B // L)
            def _grp(g):
                dg = [dir_v[t, k, pl.ds(g * L, L)] for k in range(3)]
                for j in range(L):
                    e = g * L + j
                    d = [_splat(dg[k], j) for k in range(3)]
                    for v in range(W // L):
                        sl = pl.ds(v * L, L)
                        xwm = wm_v[t, e, sl] * xg_v[tg, e, sl]
                        xwh = wh_v[t, e, sl] * xg_v[tg, e, pl.ds(W + v * L, L)]
                        for k in range(3):
                            msg_v[tm, e, k, sl] = (xwm * d[k]
                                                   + xwh * mu_v[tg, e, k, sl])

            start_scatter(b)

        wait_scatter(NBLK - 2)
        wait_scatter(NBLK - 1)
        plsc.subcore_barrier()
        pltpu.sync_copy(acc.at[pl.ds(r0, ROWS)],
                        muout_hbm.at[pl.ds(r0, ROWS), :, c, p])


def kernel(q, mu, Wij, dir_ij, idx_i, idx_j, n_atoms, W1, b1, W2, b2):
    q2 = q.reshape(N, F)
    wij2 = Wij.reshape(E, 3 * F)
    idx_i = (idx_i.astype(jnp.int32) % jnp.int32(n_atoms))
    idx_j = idx_j.astype(jnp.int32)
    dirT = dir_ij.T

    xa, xb0, xb1, qh, mq0, mq1 = _mlp(q2, mu, W1, b1.reshape(1, F), W2,
                                      b2.reshape(1, 3 * F))

    q_out = _dq_phase(wij2, xa, qh, idx_i, idx_j)
    mo = _dmu_kernel(wij2, xb0, mq0, xb1, mq1, dirT, idx_i, idx_j)
    return (q_out.reshape(N, 1, F), mo.reshape(N, 3, F))


# speedup vs baseline: 1.0666x; 1.0666x over previous
import functools

import jax
import jax.numpy as jnp
from jax import lax
from jax.experimental import pallas as pl
from jax.experimental.pallas import tpu as pltpu
from jax.experimental.pallas import tpu_sc as plsc

N = 10000
E = 320000
F = 128
H = F // 2
NC = 2
NS = 16
L = 16
EPT = E // NS
B = 80
NBLK = EPT // B
ROWS = N // NS

_mesh = plsc.VectorSubcoreMesh(core_axis_name="c", subcore_axis_name="s")

_SPLAT_DNUMS = lax.GatherDimensionNumbers(
    offset_dims=(), collapsed_slice_dims=(0,), start_index_map=(0,))


def _splat(vec, j):
    idx = jnp.full((L, 1), j, jnp.int32)
    return lax.gather(vec, idx, _SPLAT_DNUMS, (1,),
                      mode=lax.GatherScatterMode.PROMISE_IN_BOUNDS)


W = H // 2


def _mlp_body(q_ref, mu_ref, w1_ref, b1_ref, w2_ref, b2_ref,
              xa_ref, xb0_ref, xb1_ref, qh_ref, mq0_ref, mq1_ref):
    h = jnp.dot(q_ref[...], w1_ref[...], preferred_element_type=jnp.float32)
    h = h + b1_ref[...]
    h = h * lax.logistic(h)
    x = jnp.dot(h, w2_ref[...], preferred_element_type=jnp.float32)
    x = x + b2_ref[...]
    for c in range(NC):
        xa_ref[c] = x[:, c * H:(c + 1) * H]
        qh_ref[c] = q_ref[:, c * H:(c + 1) * H]
        for p, (xb_ref, mq_ref) in enumerate(((xb0_ref, mq0_ref),
                                              (xb1_ref, mq1_ref))):
            c0 = c * H + p * W
            xb_ref[c] = jnp.concatenate(
                [x[:, F + c0:F + c0 + W], x[:, 2 * F + c0:2 * F + c0 + W]],
                axis=1)
            mq_ref[c] = mu_ref[:, :, c0:c0 + W]


def _mlp(q2, mu, W1, b1, W2, b2):
    blk = 1000
    grid = N // blk
    return pl.pallas_call(
        _mlp_body,
        grid=(grid,),
        in_specs=[
            pl.BlockSpec((blk, F), lambda i: (i, 0)),
            pl.BlockSpec((blk, 3, F), lambda i: (i, 0, 0)),
            pl.BlockSpec((F, F), lambda i: (0, 0)),
            pl.BlockSpec((1, F), lambda i: (0, 0)),
            pl.BlockSpec((F, 3 * F), lambda i: (0, 0)),
            pl.BlockSpec((1, 3 * F), lambda i: (0, 0)),
        ],
        out_specs=[
            pl.BlockSpec((NC, blk, H), lambda i: (0, i, 0)),
            pl.BlockSpec((NC, blk, 2 * W), lambda i: (0, i, 0)),
            pl.BlockSpec((NC, blk, 2 * W), lambda i: (0, i, 0)),
            pl.BlockSpec((NC, blk, H), lambda i: (0, i, 0)),
            pl.BlockSpec((NC, blk, 3, W), lambda i: (0, i, 0, 0)),
            pl.BlockSpec((NC, blk, 3, W), lambda i: (0, i, 0, 0)),
        ],
        out_shape=[
            jax.ShapeDtypeStruct((NC, N, H), jnp.float32),
            jax.ShapeDtypeStruct((NC, N, 2 * W), jnp.float32),
            jax.ShapeDtypeStruct((NC, N, 2 * W), jnp.float32),
            jax.ShapeDtypeStruct((NC, N, H), jnp.float32),
            jax.ShapeDtypeStruct((NC, N, 3, W), jnp.float32),
            jax.ShapeDtypeStruct((NC, N, 3, W), jnp.float32),
        ],
    )(q2, mu, W1, b1, W2, b2)


NBUF = 3


@functools.partial(
    pl.kernel,
    out_type=jax.ShapeDtypeStruct((N, F), jnp.float32),
    mesh=_mesh,
    compiler_params=pltpu.CompilerParams(use_tc_tiling_on_sc=False),
    scratch_types=[
        pltpu.VMEM_SHARED((N, H), jnp.float32),
        pltpu.VMEM((4, B), jnp.int32),
        pltpu.VMEM((NBUF, B), jnp.int32),
        pltpu.VMEM((NBUF, B, H), jnp.float32),
        pltpu.VMEM((2, B, H), jnp.float32),
        pltpu.VMEM((NBUF, B, H), jnp.float32),
        pltpu.SemaphoreType.DMA((NBUF,)),
        pltpu.SemaphoreType.DMA((2,)),
        pltpu.SemaphoreType.DMA((4,)),
    ],
)
def _dq_phase(wij_hbm, xa_hbm, qh_hbm, idxi_hbm, idxj_hbm, qout_hbm,
              acc, idxi_v, idxj_v, w_v, xg_v, msg_v, lsem, gsem, ssem):
    c = lax.axis_index("c")
    s = lax.axis_index("s")
    r0 = s * ROWS
    pltpu.sync_copy(qh_hbm.at[c].at[pl.ds(r0, ROWS)], acc.at[pl.ds(r0, ROWS)])
    plsc.subcore_barrier()

    e_base = s * EPT
    c0 = c * H

    def lin_copies(b):
        e0 = e_base + b * B
        t = lax.rem(b, NBUF)
        return t, [
            (idxi_hbm.at[pl.ds(e0, B)], idxi_v.at[lax.rem(b, 4)]),
            (idxj_hbm.at[pl.ds(e0, B)], idxj_v.at[t]),
            (wij_hbm.at[pl.ds(e0, B), pl.ds(c0, H)], w_v.at[t]),
        ]

    def start_linear(b):
        t, copies = lin_copies(b)
        for src, dst in copies:
            pltpu.async_copy(src, dst, lsem.at[t])

    def wait_linear(b):
        t, copies = lin_copies(b)
        for src, dst in copies:
            pltpu.make_async_copy(src, dst, lsem.at[t]).wait()

    def start_gather(b):
        pltpu.async_copy(xa_hbm.at[c].at[idxj_v.at[lax.rem(b, NBUF)]],
                         xg_v.at[lax.rem(b, 2)], gsem.at[lax.rem(b, 2)])

    def wait_gather(b):
        pltpu.make_async_copy(xa_hbm.at[c].at[idxj_v.at[lax.rem(b, NBUF)]],
                              xg_v.at[lax.rem(b, 2)],
                              gsem.at[lax.rem(b, 2)]).wait()

    def start_scatter(b):
        pltpu.async_copy(msg_v.at[lax.rem(b, NBUF)],
                         acc.at[idxi_v.at[lax.rem(b, 4)]],
                         ssem.at[lax.rem(b, 4)], add=True)

    def wait_scatter(b):
        pltpu.make_async_copy(msg_v.at[lax.rem(b, NBUF)],
                              acc.at[idxi_v.at[lax.rem(b, 4)]],
                              ssem.at[lax.rem(b, 4)]).wait()

    start_linear(0)
    start_linear(1)
    wait_linear(0)
    start_gather(0)

    @pl.loop(0, NBLK)
    def _blk(b):
        t = lax.rem(b, NBUF)
        tg = lax.rem(b, 2)

        @pl.when(b >= 2)
        def _():
            wait_scatter(b - 2)

        @pl.when(b + 2 < NBLK)
        def _():
            start_linear(b + 2)

        @pl.when(b + 1 < NBLK)
        def _():
            wait_linear(b + 1)
            start_gather(b + 1)

        wait_gather(b)

        @plsc.parallel_loop(0, B)
        def _edge(e):
            for v in range(H // L):
                sl = pl.ds(v * L, L)
                msg_v[t, e, sl] = w_v[t, e, sl] * xg_v[tg, e, sl]

        start_scatter(b)

    wait_scatter(NBLK - 2)
    wait_scatter(NBLK - 1)
    plsc.subcore_barrier()
    pltpu.sync_copy(acc.at[pl.ds(r0, ROWS)],
                    qout_hbm.at[pl.ds(r0, ROWS), pl.ds(c0, H)])


def _make_dmu_phase(p):
    @functools.partial(
        pl.kernel,
        out_type=jax.ShapeDtypeStruct((N, 3, NC, W), jnp.float32),
        mesh=_mesh,
        compiler_params=pltpu.CompilerParams(use_tc_tiling_on_sc=False),
        scratch_types=[
            pltpu.VMEM_SHARED((N, 3, W), jnp.float32),
            pltpu.VMEM((4, B), jnp.int32),
            pltpu.VMEM((NBUF, B), jnp.int32),
            pltpu.VMEM((NBUF, B, W), jnp.float32),
            pltpu.VMEM((NBUF, B, W), jnp.float32),
            pltpu.VMEM((2, B, 2 * W), jnp.float32),
            pltpu.VMEM((2, B, 3, W), jnp.float32),
            pltpu.VMEM((NBUF, B, 3, W), jnp.float32),
            pltpu.VMEM((NBUF, 3, B), jnp.float32),
            pltpu.SemaphoreType.DMA((NBUF,)),
            pltpu.SemaphoreType.DMA((2,)),
            pltpu.SemaphoreType.DMA((4,)),
        ],
    )
    def _dmu_phase(wij_hbm, xbp_hbm, muq_hbm, dir_hbm, idxi_hbm, idxj_hbm,
                   muout_hbm, acc, idxi_v, idxj_v, wm_v, wh_v, xg_v, mu_v,
                   msg_v, dir_v, lsem, gsem, ssem):
        c = lax.axis_index("c")
        s = lax.axis_index("s")
        r0 = s * ROWS
        pltpu.sync_copy(muq_hbm.at[c].at[pl.ds(r0, ROWS)], acc.at[pl.ds(r0, ROWS)])
        plsc.subcore_barrier()

        e_base = s * EPT
        c0 = c * H + p * W

        def lin_copies(b):
            e0 = e_base + b * B
            t = lax.rem(b, NBUF)
            ti = lax.rem(b, 4)
            return t, [
                (idxi_hbm.at[pl.ds(e0, B)], idxi_v.at[ti]),
                (idxj_hbm.at[pl.ds(e0, B)], idxj_v.at[t]),
                (wij_hbm.at[pl.ds(e0, B), pl.ds(F + c0, W)], wm_v.at[t]),
                (wij_hbm.at[pl.ds(e0, B), pl.ds(2 * F + c0, W)], wh_v.at[t]),
                (dir_hbm.at[:, pl.ds(e0, B)], dir_v.at[t]),
            ]

        def start_linear(b):
            t, copies = lin_copies(b)
            for src, dst in copies:
                pltpu.async_copy(src, dst, lsem.at[t])

        def wait_linear(b):
            t, copies = lin_copies(b)
            for src, dst in copies:
                pltpu.make_async_copy(src, dst, lsem.at[t]).wait()

        def gat_copies(b):
            t = lax.rem(b, NBUF)
            tg = lax.rem(b, 2)
            return tg, [
                (xbp_hbm.at[c].at[idxj_v.at[t]], xg_v.at[tg]),
                (muq_hbm.at[c].at[idxj_v.at[t]], mu_v.at[tg]),
            ]

        def start_gather(b):
            tg, copies = gat_copies(b)
            for src, dst in copies:
                pltpu.async_copy(src, dst, gsem.at[tg])

        def wait_gather(b):
            tg, copies = gat_copies(b)
            for src, dst in copies:
                pltpu.make_async_copy(src, dst, gsem.at[tg]).wait()

        def start_scatter(b):
            pltpu.async_copy(msg_v.at[lax.rem(b, NBUF)],
                             acc.at[idxi_v.at[lax.rem(b, 4)]],
                             ssem.at[lax.rem(b, 4)], add=True)

        def wait_scatter(b):
            pltpu.make_async_copy(msg_v.at[lax.rem(b, NBUF)],
                                  acc.at[idxi_v.at[lax.rem(b, 4)]],
                                  ssem.at[lax.rem(b, 4)]).wait()

        start_linear(0)
        start_linear(1)
        wait_linear(0)
        start_gather(0)

        @pl.loop(0, NBLK)
        def _blk(b):
            t = lax.rem(b, NBUF)
            tg = lax.rem(b, 2)
            tm = t

            @pl.when(b >= 2)
            def _():
                wait_scatter(b - 2)

            @pl.when(b + 2 < NBLK)
            def _():
                start_linear(b + 2)

            @pl.when(b + 1 < NBLK)
            def _():
                wait_linear(b + 1)
                start_gather(b + 1)

            wait_gather(b)

            @plsc.parallel_loop(0, B // L)
            def _grp(g):
                dg = [dir_v[t, k, pl.ds(g * L, L)] for k in range(3)]
                for j in range(L):
                    e = g * L + j
                    d = [_splat(dg[k], j) for k in range(3)]
                    for v in range(W // L):
                        sl = pl.ds(v * L, L)
                        xwm = wm_v[t, e, sl] * xg_v[tg, e, sl]
                        xwh = wh_v[t, e, sl] * xg_v[tg, e, pl.ds(W + v * L, L)]
                        for k in range(3):
                            msg_v[tm, e, k, sl] = (xwm * d[k]
                                                   + xwh * mu_v[tg, e, k, sl])

            start_scatter(b)

        wait_scatter(NBLK - 2)
        wait_scatter(NBLK - 1)
        plsc.subcore_barrier()
        pltpu.sync_copy(acc.at[pl.ds(r0, ROWS)],
                        muout_hbm.at[pl.ds(r0, ROWS), :, c])

    return _dmu_phase


_dmu_phases = [_make_dmu_phase(0), _make_dmu_phase(1)]


def kernel(q, mu, Wij, dir_ij, idx_i, idx_j, n_atoms, W1, b1, W2, b2):
    q2 = q.reshape(N, F)
    wij2 = Wij.reshape(E, 3 * F)
    idx_i = (idx_i.astype(jnp.int32) % jnp.int32(n_atoms))
    idx_j = idx_j.astype(jnp.int32)
    dirT = dir_ij.T

    xa, xb0, xb1, qh, mq0, mq1 = _mlp(q2, mu, W1, b1.reshape(1, F), W2,
                                      b2.reshape(1, 3 * F))

    q_out = _dq_phase(wij2, xa, qh, idx_i, idx_j)

    mu_outs = [
        _dmu_phases[0](wij2, xb0, mq0, dirT, idx_i, idx_j),
        _dmu_phases[1](wij2, xb1, mq1, dirT, idx_i, idx_j),
    ]

    mo = jnp.stack(mu_outs)
    mu_out = mo.transpose(1, 2, 3, 0, 4).reshape(N, 3, F)
    return (q_out.reshape(N, 1, F), mu_out)
